# Initial kernel scaffold; baseline (speedup 1.0000x reference)
import functools

import jax
import jax.numpy as jnp
from jax import lax
from jax.experimental import pallas as pl
from jax.experimental.pallas import tpu as pltpu
from jax.experimental.pallas import tpu_sc as plsc

N = 10000
E = 320000
D = 256
H = 128
NC = 2
NS = 16

EPT = E // NS
K = 128
NCHUNK = -(-EPT // K)
EPT_PAD = NCHUNK * K
R_AGG = 10016
ZROWS = R_AGG // NS
OROWS = N // NS

EPW = E // (NC * NS)
NCH_D = -(-EPW // K)
EPW_PAD = NCH_D * K
R_DEG = 10240
ZROWS_D = R_DEG // NS

BN = 2000
GRID = N // BN

_mesh = plsc.VectorSubcoreMesh(core_axis_name="c", subcore_axis_name="s")


@functools.partial(
    pl.kernel,
    out_type=jax.ShapeDtypeStruct((NC, R_DEG, 8), jnp.float32),
    mesh=_mesh,
    scratch_types=[
        pltpu.VMEM((NCH_D, K), jnp.int32),
        pltpu.VMEM((K, 8), jnp.float32),
        pltpu.VMEM_SHARED((R_DEG, 8), jnp.float32),
    ],
)
def _deg_kernel(dst_hbm, ones_hbm, zeros_hbm, out_hbm, dst_v, ones_v, acc):
    c = lax.axis_index("c")
    s = lax.axis_index("s")
    pltpu.sync_copy(dst_hbm.at[c, s], dst_v)
    pltpu.sync_copy(ones_hbm, ones_v)
    pltpu.sync_copy(zeros_hbm, acc.at[pl.ds(s * ZROWS_D, ZROWS_D)])
    plsc.subcore_barrier()

    @pl.loop(0, NCH_D)
    def _(i):
        pltpu.sync_copy(ones_v, acc.at[dst_v.at[i]], add=True)

    plsc.subcore_barrier()
    pltpu.sync_copy(acc.at[pl.ds(s * ZROWS_D, ZROWS_D)],
                    out_hbm.at[c, pl.ds(s * ZROWS_D, ZROWS_D)])


@functools.partial(
    pl.kernel,
    out_type=jax.ShapeDtypeStruct((NC, N, H), jnp.float32),
    mesh=_mesh,
    scratch_types=[
        pltpu.VMEM((NCHUNK, K), jnp.int32),
        pltpu.VMEM((NCHUNK, K), jnp.int32),
        pltpu.VMEM((K, H), jnp.float32),
        pltpu.VMEM((K, H), jnp.float32),
        pltpu.VMEM_SHARED((R_AGG, H), jnp.float32),
        pltpu.SemaphoreType.DMA,
        pltpu.SemaphoreType.DMA,
    ],
)
def _agg_kernel(table_hbm, src_hbm, dst_hbm, zeros_hbm, out_hbm,
                src_v, dst_v, buf0, buf1, acc, sem0, sem1):
    c = lax.axis_index("c")
    s = lax.axis_index("s")
    pltpu.sync_copy(src_hbm.at[c, s], src_v)
    pltpu.sync_copy(dst_hbm.at[s], dst_v)
    pltpu.sync_copy(zeros_hbm, acc.at[pl.ds(s * ZROWS, ZROWS)])
    plsc.subcore_barrier()

    bufs = (buf0, buf1)
    sems = (sem0, sem1)
    pltpu.async_copy(table_hbm.at[src_v.at[0]], buf0, sem0)
    pltpu.async_copy(table_hbm.at[src_v.at[1]], buf1, sem1)

    @pl.loop(0, NCHUNK - 1, step=2)
    def _(i):
        for b in (0, 1):
            ch = i + b
            pltpu.make_async_copy(table_hbm.at[src_v.at[0]], bufs[b],
                                  sems[b]).wait()
            pltpu.sync_copy(bufs[b], acc.at[dst_v.at[ch]], add=True)

            @pl.when(ch + 2 < NCHUNK)
            def _():
                pltpu.async_copy(table_hbm.at[src_v.at[ch + 2]], bufs[b],
                                 sems[b])

    pltpu.make_async_copy(table_hbm.at[src_v.at[0]], buf0, sem0).wait()
    pltpu.sync_copy(buf0, acc.at[dst_v.at[NCHUNK - 1]], add=True)

    plsc.subcore_barrier()
    pltpu.sync_copy(acc.at[pl.ds(s * OROWS, OROWS)],
                    out_hbm.at[c, pl.ds(s * OROWS, OROWS)])


def _dinv_block(deg_ref, i):
    dd = deg_ref[0, pl.ds(i * BN, BN)] + deg_ref[1, pl.ds(i * BN, BN)] + 1.0
    return lax.rsqrt(dd)[:, None]


def _mm0_body(x_ref, w_ref, deg_ref, g_ref):
    i = pl.program_id(0)
    dinv = _dinv_block(deg_ref, i)
    u = jnp.dot(x_ref[...], w_ref[...], preferred_element_type=jnp.float32)
    g = u * dinv
    g_ref[0] = g[:, :H]
    g_ref[1] = g[:, H:]


def _mid_body(agg_ref, g_ref, deg_ref, b_ref, w_ref, out_ref):
    i = pl.program_id(0)
    dinv = _dinv_block(deg_ref, i)
    h0 = jnp.maximum((agg_ref[0] + g_ref[0]) * dinv + b_ref[0, :H], 0.0)
    h1 = jnp.maximum((agg_ref[1] + g_ref[1]) * dinv + b_ref[0, H:], 0.0)
    u = (jnp.dot(h0, w_ref[:H, :], preferred_element_type=jnp.float32)
         + jnp.dot(h1, w_ref[H:, :], preferred_element_type=jnp.float32))
    g = u * dinv
    out_ref[0] = g[:, :H]
    out_ref[1] = g[:, H:]


def _fin_body(agg_ref, g_ref, deg_ref, b_ref, out_ref):
    i = pl.program_id(0)
    dinv = _dinv_block(deg_ref, i)
    h0 = jnp.maximum((agg_ref[0] + g_ref[0]) * dinv + b_ref[0, :H], 0.0)
    h1 = jnp.maximum((agg_ref[1] + g_ref[1]) * dinv + b_ref[0, H:], 0.0)
    out_ref[...] = jnp.concatenate([h0, h1], axis=1)


_spec_g = pl.BlockSpec((NC, BN, H), lambda i: (0, i, 0))
_spec_deg = pl.BlockSpec((NC, R_DEG), lambda i: (0, 0))
_spec_w = pl.BlockSpec((D, D), lambda i: (0, 0))
_spec_b = pl.BlockSpec((1, D), lambda i: (0, 0))

_mm0 = pl.pallas_call(
    _mm0_body,
    grid=(GRID,),
    in_specs=[pl.BlockSpec((BN, D), lambda i: (i, 0)), _spec_w, _spec_deg],
    out_specs=_spec_g,
    out_shape=jax.ShapeDtypeStruct((NC, N, H), jnp.float32),
)

_mid = pl.pallas_call(
    _mid_body,
    grid=(GRID,),
    in_specs=[_spec_g, _spec_g, _spec_deg, _spec_b, _spec_w],
    out_specs=_spec_g,
    out_shape=jax.ShapeDtypeStruct((NC, N, H), jnp.float32),
)

_fin = pl.pallas_call(
    _fin_body,
    grid=(GRID,),
    in_specs=[_spec_g, _spec_g, _spec_deg, _spec_b],
    out_specs=pl.BlockSpec((BN, D), lambda i: (i, 0)),
    out_shape=jax.ShapeDtypeStruct((N, D), jnp.float32),
)


@jax.jit
def _run(x, edge_index, W0, b0, W1, b1, W2, b2):
    ei = edge_index.astype(jnp.int32)
    src, dst = ei[0], ei[1]

    src_t = src.reshape(NS, EPT)
    dst_t = dst.reshape(NS, EPT)
    pad = EPT_PAD - EPT
    src_p = jnp.concatenate(
        [src_t, jnp.zeros((NS, pad), jnp.int32)], axis=1).reshape(NS, NCHUNK, K)
    dst_p = jnp.concatenate(
        [dst_t, jnp.full((NS, pad), N, jnp.int32)], axis=1).reshape(NS, NCHUNK, K)
    src_cs = jnp.stack([src_p, src_p + N])

    pad_d = EPW_PAD - EPW
    dst_d = jnp.concatenate(
        [dst.reshape(NC, NS, EPW),
         jnp.full((NC, NS, pad_d), N, jnp.int32)], axis=2
    ).reshape(NC, NS, NCH_D, K)

    ones8 = jnp.ones((K, 8), jnp.float32)
    zeros_d = jnp.zeros((ZROWS_D, 8), jnp.float32)
    zeros_a = jnp.zeros((ZROWS, H), jnp.float32)

    deg8 = _deg_kernel(dst_d, ones8, zeros_d)
    deg2 = deg8[:, :, 0]

    b0r = b0.reshape(1, D)
    b1r = b1.reshape(1, D)
    b2r = b2.reshape(1, D)

    g1 = _mm0(x, W0, deg2)
    a1 = _agg_kernel(g1.reshape(NC * N, H), src_cs, dst_p, zeros_a)
    g2 = _mid(a1, g1, deg2, b0r, W1)
    a2 = _agg_kernel(g2.reshape(NC * N, H), src_cs, dst_p, zeros_a)
    g3 = _mid(a2, g2, deg2, b1r, W2)
    a3 = _agg_kernel(g3.reshape(NC * N, H), src_cs, dst_p, zeros_a)
    return _fin(a3, g3, deg2, b2r)


def kernel(x, edge_index, W0, b0, W1, b1, W2, b2):
    return _run(x, edge_index, W0, b0, W1, b1, W2, b2)


# speedup vs baseline: 7.3215x; 7.3215x over previous
import functools

import jax
import jax.numpy as jnp
from jax import lax
from jax.experimental import pallas as pl
from jax.experimental.pallas import tpu as pltpu
from jax.experimental.pallas import tpu_sc as plsc

N = 10000
E = 320000
D = 256
H = 128
NC = 2
NS = 16

EPT = E // NS
K = 128
G = 8
NGRP = 20
NCHUNK = NGRP * G
EPT_PAD = NCHUNK * K
NP = 10240
R_AGG = NP
ZROWS = R_AGG // NS

EPW = E // (NC * NS)
NCH_D = -(-EPW // K)
EPW_PAD = NCH_D * K
R_DEG = 10240
ZROWS_D = R_DEG // NS

BN = 2048
GRID = NP // BN

_mesh = plsc.VectorSubcoreMesh(core_axis_name="c", subcore_axis_name="s")


@functools.partial(
    pl.kernel,
    out_type=jax.ShapeDtypeStruct((NC, R_AGG, H), jnp.float32),
    mesh=_mesh,
    scratch_types=[
        pltpu.VMEM((2, G, K), jnp.int32),
        pltpu.VMEM((2, G, K), jnp.int32),
        pltpu.VMEM((K, H), jnp.float32),
        pltpu.VMEM((K, H), jnp.float32),
        pltpu.VMEM_SHARED((R_AGG, H), jnp.float32),
        pltpu.SemaphoreType.DMA,
        pltpu.SemaphoreType.DMA,
        pltpu.SemaphoreType.DMA,
        pltpu.SemaphoreType.DMA,
    ],
)
def _agg_kernel(table_hbm, src_hbm, dst_hbm, zeros_hbm, out_hbm,
                src_v, dst_v, buf0, buf1, acc, sem0, sem1, isem_s, isem_d):
    c = lax.axis_index("c")
    s = lax.axis_index("s")
    pltpu.sync_copy(src_hbm.at[c, s, 0], src_v.at[0])
    pltpu.sync_copy(dst_hbm.at[s, 0], dst_v.at[0])
    pltpu.async_copy(src_hbm.at[c, s, 1], src_v.at[1], isem_s)
    pltpu.async_copy(dst_hbm.at[s, 1], dst_v.at[1], isem_d)
    pltpu.sync_copy(zeros_hbm, acc.at[pl.ds(s * ZROWS, ZROWS)])

    bufs = (buf0, buf1)
    sems = (sem0, sem1)
    pltpu.async_copy(table_hbm.at[src_v.at[0, 0]], buf0, sem0)
    pltpu.async_copy(table_hbm.at[src_v.at[0, 1]], buf1, sem1)
    plsc.subcore_barrier()

    def _wait_gather(b):
        pltpu.make_async_copy(table_hbm.at[src_v.at[0, 0]], bufs[b],
                              sems[b]).wait()

    @pl.loop(0, NGRP)
    def _(g):
        ib = g % 2
        for j in range(G):
            b = j % 2
            _wait_gather(b)
            pltpu.sync_copy(bufs[b], acc.at[dst_v.at[ib, j]], add=True)
            if j == G - 2:
                @pl.when(g < NGRP - 1)
                def _():
                    pltpu.make_async_copy(src_hbm.at[c, s, 0], src_v.at[0],
                                          isem_s).wait()
                    pltpu.make_async_copy(dst_hbm.at[s, 0], dst_v.at[0],
                                          isem_d).wait()
            if j < G - 2:
                pltpu.async_copy(table_hbm.at[src_v.at[ib, j + 2]],
                                 bufs[b], sems[b])
            else:
                @pl.when(g < NGRP - 1)
                def _():
                    pltpu.async_copy(table_hbm.at[src_v.at[1 - ib, j + 2 - G]],
                                     bufs[b], sems[b])
        @pl.when(g < NGRP - 2)
        def _():
            pltpu.async_copy(src_hbm.at[c, s, g + 2], src_v.at[ib], isem_s)
            pltpu.async_copy(dst_hbm.at[s, g + 2], dst_v.at[ib], isem_d)

    plsc.subcore_barrier()
    pltpu.sync_copy(acc.at[pl.ds(s * ZROWS, ZROWS)],
                    out_hbm.at[c, pl.ds(s * ZROWS, ZROWS)])


def _dinv_block(deg_ref, i):
    off = pl.multiple_of(i * BN, 128)
    dd = deg_ref[0, pl.ds(off, BN)] + deg_ref[1, pl.ds(off, BN)] + 1.0
    return lax.rsqrt(dd)[:, None]


def _mm0_body(x_ref, w_ref, deg_ref, g_ref):
    i = pl.program_id(0)
    dinv = _dinv_block(deg_ref, i)
    u = jnp.dot(x_ref[...], w_ref[...], preferred_element_type=jnp.float32)
    g = u * dinv
    g_ref[0] = g[:, :H]
    g_ref[1] = g[:, H:]


def _mid_body(agg_ref, g_ref, deg_ref, b_ref, w_ref, out_ref):
    i = pl.program_id(0)
    dinv = _dinv_block(deg_ref, i)
    h0 = jnp.maximum((agg_ref[0] + g_ref[0]) * dinv + b_ref[0, :H], 0.0)
    h1 = jnp.maximum((agg_ref[1] + g_ref[1]) * dinv + b_ref[0, H:], 0.0)
    h = jnp.concatenate([h0, h1], axis=1)
    u = jnp.dot(h, w_ref[...], preferred_element_type=jnp.float32)
    g = u * dinv
    out_ref[0] = g[:, :H]
    out_ref[1] = g[:, H:]


def _fin_body(agg_ref, g_ref, deg_ref, b_ref, out_ref):
    i = pl.program_id(0)
    dinv = _dinv_block(deg_ref, i)
    h0 = jnp.maximum((agg_ref[0] + g_ref[0]) * dinv + b_ref[0, :H], 0.0)
    h1 = jnp.maximum((agg_ref[1] + g_ref[1]) * dinv + b_ref[0, H:], 0.0)
    out_ref[...] = jnp.concatenate([h0, h1], axis=1)


_spec_g = pl.BlockSpec((NC, BN, H), lambda i: (0, i, 0))
_spec_deg = pl.BlockSpec((NC, R_DEG), lambda i: (0, 0))
_spec_w = pl.BlockSpec((D, D), lambda i: (0, 0))
_spec_b = pl.BlockSpec((1, D), lambda i: (0, 0))

_mm0 = pl.pallas_call(
    _mm0_body,
    grid=(GRID,),
    in_specs=[pl.BlockSpec((BN, D), lambda i: (i, 0)), _spec_w, _spec_deg],
    out_specs=_spec_g,
    out_shape=jax.ShapeDtypeStruct((NC, NP, H), jnp.float32),
)

_mid = pl.pallas_call(
    _mid_body,
    grid=(GRID,),
    in_specs=[_spec_g, _spec_g, _spec_deg, _spec_b, _spec_w],
    out_specs=_spec_g,
    out_shape=jax.ShapeDtypeStruct((NC, NP, H), jnp.float32),
)

_fin = pl.pallas_call(
    _fin_body,
    grid=(GRID,),
    in_specs=[_spec_g, _spec_g, _spec_deg, _spec_b],
    out_specs=pl.BlockSpec((BN, D), lambda i: (i, 0)),
    out_shape=jax.ShapeDtypeStruct((NP, D), jnp.float32),
)


@jax.jit
def _run(x, edge_index, W0, b0, W1, b1, W2, b2):
    ei = edge_index.astype(jnp.int32)
    src, dst = ei[0], ei[1]

    src_t = src.reshape(NS, EPT)
    dst_t = dst.reshape(NS, EPT)
    pad = EPT_PAD - EPT
    src_p = jnp.concatenate(
        [src_t, jnp.zeros((NS, pad), jnp.int32)],
        axis=1).reshape(NS, NGRP, G, K)
    dst_p = jnp.concatenate(
        [dst_t, jnp.full((NS, pad), N, jnp.int32)],
        axis=1).reshape(NS, NGRP, G, K)
    src_cs = jnp.stack([src_p, src_p + NP])

    zeros_a = jnp.zeros((ZROWS, H), jnp.float32)

    ones_tab = jnp.ones((NC * NP, H), jnp.float32)
    degA = _agg_kernel(ones_tab, src_cs, dst_p, zeros_a)
    deg2 = jnp.stack([degA[0, :, 0], jnp.zeros((NP,), jnp.float32)])
    xp = jnp.pad(x, ((0, NP - N), (0, 0)))

    b0r = b0.reshape(1, D)
    b1r = b1.reshape(1, D)
    b2r = b2.reshape(1, D)

    g1 = _mm0(xp, W0, deg2)
    a1 = _agg_kernel(g1.reshape(NC * NP, H), src_cs, dst_p, zeros_a)
    g2 = _mid(a1, g1, deg2, b0r, W1)
    a2 = _agg_kernel(g2.reshape(NC * NP, H), src_cs, dst_p, zeros_a)
    g3 = _mid(a2, g2, deg2, b1r, W2)
    a3 = _agg_kernel(g3.reshape(NC * NP, H), src_cs, dst_p, zeros_a)
    return _fin(a3, g3, deg2, b2r)[:N]


def kernel(x, edge_index, W0, b0, W1, b1, W2, b2):
    return _run(x, edge_index, W0, b0, W1, b1, W2, b2)


# speedup vs baseline: 8.6751x; 1.1849x over previous
import functools

import jax
import jax.numpy as jnp
from jax import lax
from jax.experimental import pallas as pl
from jax.experimental.pallas import tpu as pltpu
from jax.experimental.pallas import tpu_sc as plsc

N = 10000
E = 320000
D = 256
H = 128
NC = 2
NS = 16

EPT = E // NS
K = 64
G = 8
NGRP = 40
NCHUNK = NGRP * G
EPT_PAD = NCHUNK * K
B = 4
NP = 10240
R_AGG = NP
ZROWS = R_AGG // NS
NGRP_D = NGRP // NC

BN = 2048
GRID = NP // BN

_mesh = plsc.VectorSubcoreMesh(core_axis_name="c", subcore_axis_name="s")


@functools.partial(
    pl.kernel,
    out_type=jax.ShapeDtypeStruct((NC, R_AGG, H), jnp.float32),
    mesh=_mesh,
    scratch_types=[
        pltpu.VMEM((2, G, K), jnp.int32),
        pltpu.VMEM((2, G, K), jnp.int32),
        [pltpu.VMEM((K, H), jnp.float32) for _ in range(B)],
        pltpu.VMEM_SHARED((R_AGG, H), jnp.float32),
        [pltpu.SemaphoreType.DMA for _ in range(B)],
        [pltpu.SemaphoreType.DMA for _ in range(B)],
        pltpu.SemaphoreType.DMA,
        pltpu.SemaphoreType.DMA,
    ],
)
def _agg_kernel(table_hbm, src_hbm, dst_hbm, zeros_hbm, out_hbm,
                src_v, dst_v, bufs, acc, gsems, ssems, isem_s, isem_d):
    c = lax.axis_index("c")
    s = lax.axis_index("s")
    pltpu.sync_copy(src_hbm.at[c, s, 0], src_v.at[0])
    pltpu.sync_copy(dst_hbm.at[s, 0], dst_v.at[0])
    pltpu.async_copy(src_hbm.at[c, s, 1], src_v.at[1], isem_s)
    pltpu.async_copy(dst_hbm.at[s, 1], dst_v.at[1], isem_d)
    pltpu.sync_copy(zeros_hbm, acc.at[pl.ds(s * ZROWS, ZROWS)])

    pltpu.async_copy(table_hbm.at[src_v.at[0, 0]], bufs[0], gsems[0])
    pltpu.async_copy(table_hbm.at[src_v.at[0, 1]], bufs[1], gsems[1])
    plsc.subcore_barrier()

    def _wait_gather(b):
        pltpu.make_async_copy(table_hbm.at[src_v.at[0, 0]], bufs[b],
                              gsems[b]).wait()

    def _wait_scatter(b):
        pltpu.make_async_copy(bufs[b], acc.at[dst_v.at[0, 0]],
                              ssems[b]).wait()

    @pl.loop(0, NGRP)
    def _(g):
        ib = g % 2
        for j in range(G):
            b = j % B
            b2 = (j + 2) % B
            _wait_gather(b)
            pltpu.async_copy(bufs[b], acc.at[dst_v.at[ib, j]], ssems[b],
                             add=True)
            if j == G - 2:
                @pl.when(g < NGRP - 1)
                def _():
                    pltpu.make_async_copy(src_hbm.at[c, s, 0], src_v.at[0],
                                          isem_s).wait()
                    pltpu.make_async_copy(dst_hbm.at[s, 0], dst_v.at[0],
                                          isem_d).wait()
            if j >= 2:
                _wait_scatter(b2)
            if j < G - 2:
                pltpu.async_copy(table_hbm.at[src_v.at[ib, j + 2]],
                                 bufs[b2], gsems[b2])
            else:
                @pl.when(g < NGRP - 1)
                def _():
                    pltpu.async_copy(table_hbm.at[src_v.at[1 - ib, j + 2 - G]],
                                     bufs[b2], gsems[b2])
        _wait_scatter((G - 2) % B)
        _wait_scatter((G - 1) % B)
        @pl.when(g < NGRP - 2)
        def _():
            pltpu.async_copy(src_hbm.at[c, s, g + 2], src_v.at[ib], isem_s)
            pltpu.async_copy(dst_hbm.at[s, g + 2], dst_v.at[ib], isem_d)

    plsc.subcore_barrier()
    pltpu.sync_copy(acc.at[pl.ds(s * ZROWS, ZROWS)],
                    out_hbm.at[c, pl.ds(s * ZROWS, ZROWS)])


@functools.partial(
    pl.kernel,
    out_type=jax.ShapeDtypeStruct((NC, R_AGG, H), jnp.float32),
    mesh=_mesh,
    scratch_types=[
        pltpu.VMEM((2, G, K), jnp.int32),
        pltpu.VMEM((K, H), jnp.float32),
        pltpu.VMEM_SHARED((R_AGG, H), jnp.float32),
        pltpu.SemaphoreType.DMA,
        pltpu.SemaphoreType.DMA,
    ],
)
def _deg_kernel(ones_hbm, dst_hbm, zeros_hbm, out_hbm,
                dst_v, ones_v, acc, ssem, isem):
    c = lax.axis_index("c")
    s = lax.axis_index("s")
    g0 = c * NGRP_D
    pltpu.sync_copy(ones_hbm, ones_v)
    pltpu.sync_copy(dst_hbm.at[s, g0], dst_v.at[0])
    pltpu.async_copy(dst_hbm.at[s, g0 + 1], dst_v.at[1], isem)
    pltpu.sync_copy(zeros_hbm, acc.at[pl.ds(s * ZROWS, ZROWS)])
    plsc.subcore_barrier()

    def _wait_scat():
        pltpu.make_async_copy(ones_v, acc.at[dst_v.at[0, 0]],
                              ssem).wait()

    @pl.loop(0, NGRP_D)
    def _(g):
        ib = g % 2
        @pl.when(g > 0)
        def _():
            pltpu.make_async_copy(dst_hbm.at[s, g0], dst_v.at[0],
                                  isem).wait()
        for j in range(G):
            pltpu.async_copy(ones_v, acc.at[dst_v.at[ib, j]], ssem, add=True)
            if j >= 4:
                _wait_scat()
        for _ in range(4):
            _wait_scat()
        @pl.when(g < NGRP_D - 2)
        def _():
            pltpu.async_copy(dst_hbm.at[s, g0 + g + 2], dst_v.at[ib], isem)

    plsc.subcore_barrier()
    pltpu.sync_copy(acc.at[pl.ds(s * ZROWS, ZROWS)],
                    out_hbm.at[c, pl.ds(s * ZROWS, ZROWS)])


def _dinv_block(deg_ref, i):
    off = pl.multiple_of(i * BN, 128)
    dd = deg_ref[0, pl.ds(off, BN)] + deg_ref[1, pl.ds(off, BN)] + 1.0
    return lax.rsqrt(dd)[:, None]


def _mm0_body(x_ref, w_ref, deg_ref, g_ref):
    i = pl.program_id(0)
    dinv = _dinv_block(deg_ref, i)
    u = jnp.dot(x_ref[...], w_ref[...], preferred_element_type=jnp.float32)
    g = u * dinv
    g_ref[0] = g[:, :H]
    g_ref[1] = g[:, H:]


def _mid_body(agg_ref, g_ref, deg_ref, b_ref, w_ref, out_ref):
    i = pl.program_id(0)
    dinv = _dinv_block(deg_ref, i)
    h0 = jnp.maximum((agg_ref[0] + g_ref[0]) * dinv + b_ref[0, :H], 0.0)
    h1 = jnp.maximum((agg_ref[1] + g_ref[1]) * dinv + b_ref[0, H:], 0.0)
    h = jnp.concatenate([h0, h1], axis=1)
    u = jnp.dot(h, w_ref[...], preferred_element_type=jnp.float32)
    g = u * dinv
    out_ref[0] = g[:, :H]
    out_ref[1] = g[:, H:]


def _fin_body(agg_ref, g_ref, deg_ref, b_ref, out_ref):
    i = pl.program_id(0)
    dinv = _dinv_block(deg_ref, i)
    h0 = jnp.maximum((agg_ref[0] + g_ref[0]) * dinv + b_ref[0, :H], 0.0)
    h1 = jnp.maximum((agg_ref[1] + g_ref[1]) * dinv + b_ref[0, H:], 0.0)
    out_ref[...] = jnp.concatenate([h0, h1], axis=1)


_spec_g = pl.BlockSpec((NC, BN, H), lambda i: (0, i, 0))
_spec_deg = pl.BlockSpec((NC, NP), lambda i: (0, 0))
_spec_w = pl.BlockSpec((D, D), lambda i: (0, 0))
_spec_b = pl.BlockSpec((1, D), lambda i: (0, 0))

_mm0 = pl.pallas_call(
    _mm0_body,
    grid=(GRID,),
    in_specs=[pl.BlockSpec((BN, D), lambda i: (i, 0)), _spec_w, _spec_deg],
    out_specs=_spec_g,
    out_shape=jax.ShapeDtypeStruct((NC, NP, H), jnp.float32),
)

_mid = pl.pallas_call(
    _mid_body,
    grid=(GRID,),
    in_specs=[_spec_g, _spec_g, _spec_deg, _spec_b, _spec_w],
    out_specs=_spec_g,
    out_shape=jax.ShapeDtypeStruct((NC, NP, H), jnp.float32),
)

_fin = pl.pallas_call(
    _fin_body,
    grid=(GRID,),
    in_specs=[_spec_g, _spec_g, _spec_deg, _spec_b],
    out_specs=pl.BlockSpec((BN, D), lambda i: (i, 0)),
    out_shape=jax.ShapeDtypeStruct((NP, D), jnp.float32),
)


@jax.jit
def _run(x, edge_index, W0, b0, W1, b1, W2, b2):
    ei = edge_index.astype(jnp.int32)
    src, dst = ei[0], ei[1]

    src_t = src.reshape(NS, EPT)
    dst_t = dst.reshape(NS, EPT)
    pad = EPT_PAD - EPT
    src_p = jnp.concatenate(
        [src_t, jnp.zeros((NS, pad), jnp.int32)],
        axis=1).reshape(NS, NGRP, G, K)
    dst_p = jnp.concatenate(
        [dst_t, jnp.full((NS, pad), N, jnp.int32)],
        axis=1).reshape(NS, NGRP, G, K)
    src_cs = jnp.stack([src_p, src_p + NP])

    zeros_a = jnp.zeros((ZROWS, H), jnp.float32)
    ones_k = jnp.ones((K, H), jnp.float32)

    degA = _deg_kernel(ones_k, dst_p, zeros_a)
    deg2 = degA[:, :, 0]
    xp = jnp.pad(x, ((0, NP - N), (0, 0)))

    b0r = b0.reshape(1, D)
    b1r = b1.reshape(1, D)
    b2r = b2.reshape(1, D)

    g1 = _mm0(xp, W0, deg2)
    a1 = _agg_kernel(g1.reshape(NC * NP, H), src_cs, dst_p, zeros_a)
    g2 = _mid(a1, g1, deg2, b0r, W1)
    a2 = _agg_kernel(g2.reshape(NC * NP, H), src_cs, dst_p, zeros_a)
    g3 = _mid(a2, g2, deg2, b1r, W2)
    a3 = _agg_kernel(g3.reshape(NC * NP, H), src_cs, dst_p, zeros_a)
    return _fin(a3, g3, deg2, b2r)[:N]


def kernel(x, edge_index, W0, b0, W1, b1, W2, b2):
    return _run(x, edge_index, W0, b0, W1, b1, W2, b2)
